# speedup vs baseline: 2.7424x; 2.7424x over previous
import jax
import jax.numpy as jnp
from jax import lax
from jax.experimental import pallas as pl
from jax.experimental.pallas import tpu as pltpu

_N = 8192
_XD = 2048
_CD = 128
_HD = 64
_E = 64
_TN = 256
_GRID = _N // _TN
_LOSS_COEF = 1e-2


def _body(x_ref, c_ref,
          ew0, eb0, ew1, eb1, ew2, eb2,
          dw0, db0, dw1, db1, dw2, db2,
          wg, w1cat, b1cat, w2v, eb2x,
          w_ref, prob_ref, z_ref, xhat_ref, gates_ref, loss_ref,
          acc_ref):
    i = pl.program_id(0)
    f32 = jnp.float32
    dot = lambda a, b: jnp.dot(a, b, preferred_element_type=f32)

    h = jax.nn.relu(dot(x_ref[...], ew0[...]) + eb0[...])
    h = jax.nn.relu(dot(h, ew1[...]) + eb1[...])
    z = dot(h, ew2[...]) + eb2[...]
    z_ref[...] = z

    h = jax.nn.relu(dot(z, dw0[...]) + db0[...])
    h = jax.nn.relu(dot(h, dw1[...]) + db1[...])
    xhat_ref[...] = dot(h, dw2[...]) + db2[...]

    logits = dot(z, wg[...])
    eio = lax.broadcasted_iota(jnp.int32, (_TN, _E), 1)
    m1 = jnp.max(logits, axis=1, keepdims=True)
    i1 = jnp.min(jnp.where(logits == m1, eio, _E), axis=1, keepdims=True)
    masked = jnp.where(eio == i1, -jnp.inf, logits)
    m2 = jnp.max(masked, axis=1, keepdims=True)
    i2 = jnp.min(jnp.where(masked == m2, eio, _E), axis=1, keepdims=True)
    e2 = jnp.exp(m2 - m1)
    denom = 1.0 + e2
    g1 = 1.0 / denom
    g2 = e2 / denom
    gates = jnp.where(eio == i1, g1, 0.0) + jnp.where(eio == i2, g2, 0.0)
    gates_ref[...] = gates

    hh = jax.nn.relu(dot(z, w1cat[...]) + b1cat[...])
    fio = lax.broadcasted_iota(jnp.int32, (_TN, _E * _HD), 1)
    fexp = lax.shift_right_logical(fio, 6)
    gx = jnp.where(fexp == i1, g1, 0.0) + jnp.where(fexp == i2, g2, 0.0)
    wout = dot(hh * gx, w2v[...]) + dot(gates, eb2x[...])
    w_ref[...] = wout

    score = jnp.sum(c_ref[...] * wout, axis=1, keepdims=True)
    prob_ref[...] = 1.0 / (1.0 + jnp.exp(-score))

    @pl.when(i == 0)
    def _init():
        acc_ref[...] = jnp.zeros_like(acc_ref)

    acc_ref[0:1, :] = acc_ref[0:1, :] + jnp.sum(gates, axis=0, keepdims=True)
    acc_ref[1:2, :] = acc_ref[1:2, :] + jnp.sum((gates > 0).astype(f32), axis=0,
                                                keepdims=True)

    @pl.when(i == _GRID - 1)
    def _loss():
        def cv2(v):
            mean = jnp.sum(v) / _E
            var = jnp.sum((v - mean) ** 2) / (_E - 1)
            return var / (mean * mean + 1e-10)
        loss_ref[0, 0] = (cv2(acc_ref[0:1, :]) + cv2(acc_ref[1:2, :])) * _LOSS_COEF


def kernel(x, c, enc_W0, enc_b0, enc_W1, enc_b1, enc_W2, enc_b2,
           dec_W0, dec_b0, dec_W1, dec_b1, dec_W2, dec_b2,
           w_gate, e_W1, e_b1, e_W2, e_b2):
    w1cat = jnp.transpose(e_W1, (1, 0, 2)).reshape(_HD, _E * _HD)
    b1cat = e_b1.reshape(1, _E * _HD)
    w2v = e_W2.reshape(_E * _HD, _CD)
    row = lambda b: b.reshape(1, -1)

    full = lambda shp: pl.BlockSpec(shp, lambda i: (0, 0))
    tok = lambda d: pl.BlockSpec((_TN, d), lambda i: (i, 0))

    out_shape = (
        jax.ShapeDtypeStruct((_N, _CD), jnp.float32),
        jax.ShapeDtypeStruct((_N, 1), jnp.float32),
        jax.ShapeDtypeStruct((_N, _HD), jnp.float32),
        jax.ShapeDtypeStruct((_N, _XD), jnp.float32),
        jax.ShapeDtypeStruct((_N, _E), jnp.float32),
        jax.ShapeDtypeStruct((1, 1), jnp.float32),
    )
    in_specs = [
        tok(_XD), tok(_CD),
        full((_XD, _HD)), full((1, _HD)), full((_HD, _HD)), full((1, _HD)),
        full((_HD, _HD)), full((1, _HD)),
        full((_HD, _HD)), full((1, _HD)), full((_HD, _HD)), full((1, _HD)),
        full((_HD, _XD)), full((1, _XD)),
        full((_HD, _E)), full((_HD, _E * _HD)), full((1, _E * _HD)),
        full((_E * _HD, _CD)), full((_E, _CD)),
    ]
    out_specs = (
        tok(_CD), tok(1), tok(_HD), tok(_XD), tok(_E),
        pl.BlockSpec((1, 1), lambda i: (0, 0), memory_space=pltpu.SMEM),
    )
    w, prob, z, xhat, gates, loss = pl.pallas_call(
        _body,
        grid=(_GRID,),
        in_specs=in_specs,
        out_specs=out_specs,
        out_shape=out_shape,
        scratch_shapes=[pltpu.VMEM((8, _E), jnp.float32)],
    )(x, c, enc_W0, row(enc_b0), enc_W1, row(enc_b1), enc_W2, row(enc_b2),
      dec_W0, row(dec_b0), dec_W1, row(dec_b1), dec_W2, row(dec_b2),
      w_gate, w1cat, b1cat, w2v, e_b2)
    return (w, prob, z, xhat, loss.reshape(()), gates)


# speedup vs baseline: 2.9796x; 1.0865x over previous
import jax
import jax.numpy as jnp
from jax import lax
from jax.experimental import pallas as pl
from jax.experimental.pallas import tpu as pltpu

_N = 8192
_XD = 2048
_CD = 128
_HD = 64
_E = 64
_TN = 512
_GRID = _N // _TN
_LOSS_COEF = 1e-2


def _body(x_ref, c_ref,
          ew0, eb0, ew1, eb1, ew2, eb2,
          dw0, db0, dw1, db1, dw2, db2,
          wg, w1cat, b1cat, w2v, eb2x, eio_ref, expand_ref,
          w_ref, prob_ref, z_ref, xhat_ref, gates_ref, loss_ref,
          acc_ref):
    i = pl.program_id(0)
    f32 = jnp.float32
    bf16 = jnp.bfloat16

    h0_t = lax.dot_general(ew0[...], x_ref[...],
                           dimension_numbers=(((0,), (1,)), ((), ())),
                           preferred_element_type=f32)
    h = jax.nn.relu(h0_t.T + eb0[...])
    h = jax.nn.relu(jnp.dot(h, ew1[...], preferred_element_type=f32) + eb1[...])
    z = jnp.dot(h, ew2[...], preferred_element_type=f32) + eb2[...]
    z_ref[...] = z

    h = jax.nn.relu(jnp.dot(z, dw0[...], preferred_element_type=f32) + db0[...])
    h = jax.nn.relu(jnp.dot(h, dw1[...], preferred_element_type=f32) + db1[...])
    xhat_ref[...] = jnp.dot(h.astype(bf16), dw2[...], preferred_element_type=f32) + db2[...]

    logits = jnp.dot(z, wg[...], preferred_element_type=f32)
    eio = eio_ref[...]
    m1 = jnp.max(logits, axis=1, keepdims=True)
    i1 = jnp.min(jnp.where(logits == m1, eio, _E), axis=1, keepdims=True)
    masked = jnp.where(eio == i1, -jnp.inf, logits)
    m2 = jnp.max(masked, axis=1, keepdims=True)
    i2 = jnp.min(jnp.where(masked == m2, eio, _E), axis=1, keepdims=True)
    e2 = jnp.exp(m2 - m1)
    denom = 1.0 + e2
    g1 = 1.0 / denom
    g2 = e2 / denom
    gates = jnp.where(eio == i1, g1, 0.0) + jnp.where(eio == i2, g2, 0.0)
    gates_ref[...] = gates

    hh = jax.nn.relu(jnp.dot(z.astype(bf16), w1cat[...], preferred_element_type=f32) +
                     b1cat[...].astype(f32)).astype(bf16)
    gx = jnp.dot(gates.astype(bf16), expand_ref[...], preferred_element_type=f32).astype(bf16)
    wout = jnp.dot(hh * gx, w2v[...], preferred_element_type=f32) + jnp.dot(gates, eb2x[...], preferred_element_type=f32)
    w_ref[...] = wout

    score = jnp.sum(c_ref[...] * wout, axis=1, keepdims=True)
    prob_ref[...] = 1.0 / (1.0 + jnp.exp(-score))

    @pl.when(i == 0)
    def _init():
        acc_ref[...] = jnp.zeros_like(acc_ref)

    acc_ref[0:1, :] = acc_ref[0:1, :] + jnp.sum(gates, axis=0, keepdims=True)
    acc_ref[1:2, :] = acc_ref[1:2, :] + jnp.sum((gates > 0).astype(f32), axis=0,
                                                keepdims=True)

    @pl.when(i == _GRID - 1)
    def _loss():
        def cv2(v):
            mean = jnp.sum(v) / _E
            var = jnp.sum((v - mean) ** 2) / (_E - 1)
            return var / (mean * mean + 1e-10)
        loss_ref[0, 0] = (cv2(acc_ref[0:1, :]) + cv2(acc_ref[1:2, :])) * _LOSS_COEF


def kernel(x, c, enc_W0, enc_b0, enc_W1, enc_b1, enc_W2, enc_b2,
           dec_W0, dec_b0, dec_W1, dec_b1, dec_W2, dec_b2,
           w_gate, e_W1, e_b1, e_W2, e_b2):
    w1cat = jnp.transpose(e_W1, (1, 0, 2)).reshape(_HD, _E * _HD).astype(jnp.bfloat16)
    b1cat = e_b1.reshape(1, _E * _HD).astype(jnp.bfloat16)
    w2v = e_W2.reshape(_E * _HD, _CD).astype(jnp.bfloat16)
    dw2b = dec_W2.astype(jnp.bfloat16)
    eio = jnp.arange(_E, dtype=jnp.int32).reshape(1, _E)
    expand = jnp.kron(jnp.eye(_E, dtype=jnp.bfloat16),
                      jnp.ones((1, _HD), dtype=jnp.bfloat16))
    row = lambda b: b.reshape(1, -1)

    full = lambda shp: pl.BlockSpec(shp, lambda i: (0, 0))
    tok = lambda d: pl.BlockSpec((_TN, d), lambda i: (i, 0))

    out_shape = (
        jax.ShapeDtypeStruct((_N, _CD), jnp.float32),
        jax.ShapeDtypeStruct((_N, 1), jnp.float32),
        jax.ShapeDtypeStruct((_N, _HD), jnp.float32),
        jax.ShapeDtypeStruct((_N, _XD), jnp.float32),
        jax.ShapeDtypeStruct((_N, _E), jnp.float32),
        jax.ShapeDtypeStruct((1, 1), jnp.float32),
    )
    in_specs = [
        tok(_XD), tok(_CD),
        full((_XD, _HD)), full((1, _HD)), full((_HD, _HD)), full((1, _HD)),
        full((_HD, _HD)), full((1, _HD)),
        full((_HD, _HD)), full((1, _HD)), full((_HD, _HD)), full((1, _HD)),
        full((_HD, _XD)), full((1, _XD)),
        full((_HD, _E)), full((_HD, _E * _HD)), full((1, _E * _HD)),
        full((_E * _HD, _CD)), full((_E, _CD)),
        full((1, _E)), full((_E, _E * _HD)),
    ]
    out_specs = (
        tok(_CD), tok(1), tok(_HD), tok(_XD), tok(_E),
        pl.BlockSpec((1, 1), lambda i: (0, 0), memory_space=pltpu.SMEM),
    )
    w, prob, z, xhat, gates, loss = pl.pallas_call(
        _body,
        grid=(_GRID,),
        in_specs=in_specs,
        out_specs=out_specs,
        out_shape=out_shape,
        scratch_shapes=[pltpu.VMEM((8, _E), jnp.float32)],
    )(x, c, enc_W0, row(enc_b0), enc_W1, row(enc_b1), enc_W2, row(enc_b2),
      dec_W0, row(dec_b0), dec_W1, row(dec_b1), dw2b, row(dec_b2),
      w_gate, w1cat, b1cat, w2v, e_b2, eio, expand)
    return (w, prob, z, xhat, loss.reshape(()), gates)
